# Initial kernel scaffold; baseline (speedup 1.0000x reference)
import jax
import jax.numpy as jnp
from jax.experimental import pallas as pl


def kernel(t, table, W, b):
    raise NotImplementedError("write your pallas kernel here")


# speedup vs baseline: 2.0129x; 2.0129x over previous
import functools

import jax
import jax.numpy as jnp
from jax import lax
from jax.experimental import pallas as pl
from jax.experimental.pallas import tpu as pltpu
from jax.experimental.pallas import tpu_sc as plsc


def _fuse_body(table_ref, w_ref, b_ref, out_ref):
    out_ref[...] = lax.dot_general(
        table_ref[...], w_ref[...],
        dimension_numbers=(((1,), (1,)), ((), ())),
        preferred_element_type=jnp.float32,
    ) + b_ref[...]


@functools.cache
def _build_gather(B, D):
    info = plsc.get_sparse_core_info()
    num_cores = info.num_cores
    NW = info.num_cores * info.num_subcores
    b_per_w = B // NW
    CHUNK = 128
    n_chunks = b_per_w // CHUNK
    assert B == NW * n_chunks * CHUNK
    mesh = plsc.VectorSubcoreMesh(core_axis_name="c", subcore_axis_name="s")

    @functools.partial(
        pl.kernel, mesh=mesh,
        out_type=jax.ShapeDtypeStruct((NW, b_per_w, D), jnp.float32),
        scratch_types=[
            pltpu.VMEM((n_chunks, CHUNK), jnp.int32),
            pltpu.VMEM((b_per_w, D), jnp.float32),
            pltpu.SemaphoreType.DMA,
        ],
    )
    def gather(fused_hbm, idx_hbm, out_hbm, idx_v, rows_v, sem):
        wid = lax.axis_index("s") * num_cores + lax.axis_index("c")
        pltpu.sync_copy(idx_hbm.at[wid], idx_v)
        copies = [
            pltpu.async_copy(fused_hbm.at[idx_v.at[j]],
                             rows_v.at[pl.ds(j * CHUNK, CHUNK)], sem)
            for j in range(n_chunks)
        ]
        for c in copies:
            c.wait()
        pltpu.sync_copy(rows_v, out_hbm.at[wid])

    return gather, NW, n_chunks, CHUNK


def kernel(t, table, W, b):
    B = t.shape[0]
    V, D = table.shape
    fused = pl.pallas_call(
        _fuse_body,
        out_shape=jax.ShapeDtypeStruct((V, D), jnp.float32),
    )(table, W, b.reshape(1, D))
    gather, NW, n_chunks, CHUNK = _build_gather(B, D)
    idx = t.reshape(NW, n_chunks, CHUNK)
    out = gather(fused, idx)
    return out.reshape(B, D)
